# baseline (speedup 1.0000x reference)
import functools

import jax
import jax.numpy as jnp
from jax import lax
from jax.experimental import pallas as pl
from jax.experimental.pallas import tpu as pltpu
from jax.experimental.pallas import tpu_sc as plsc

_B = 128
_V = 1000
_L = 16
_NCH = _B // _L
_HPAD = ((_V + _L - 1) // _L) * _L


def _tec_body(inp_hbm, tgt_hbm, out_hbm, tgt_v, idx_v, picked_v, hist_v,
              out_v, sem):
    pltpu.sync_copy(tgt_hbm, tgt_v)

    zeros = jnp.zeros((_L,), jnp.float32)
    ones = jnp.ones((_L,), jnp.float32)
    iota = lax.iota(jnp.int32, _L)
    for k in range(_NCH):
        t = tgt_v[pl.ds(k * _L, _L)]
        plsc.store_scatter(hist_v, [t], zeros)
        idx_v[pl.ds(k * _L, _L)] = t + (iota + k * _L) * _V

    cp = pltpu.async_copy(inp_hbm.at[idx_v], picked_v, sem)
    for k in range(_NCH):
        t = tgt_v[pl.ds(k * _L, _L)]
        plsc.addupdate_scatter(hist_v, [t], ones)
    cp.wait()

    num = jnp.zeros((_L,), jnp.float32)
    den = jnp.zeros((_L,), jnp.float32)
    for k in range(_NCH):
        t = tgt_v[pl.ds(k * _L, _L)]
        wt = plsc.load_gather(hist_v, [t])
        num = num + wt * picked_v[pl.ds(k * _L, _L)]
        den = den + wt

    n_v = jnp.broadcast_to(jnp.sum(num), (_L,))
    d_v = jnp.broadcast_to(jnp.sum(den), (_L,))
    out_v[...] = -(n_v / d_v)
    pltpu.sync_copy(out_v, out_hbm)


_negloss_sc = functools.partial(
    pl.kernel,
    out_type=jax.ShapeDtypeStruct((_L,), jnp.float32),
    mesh=plsc.VectorSubcoreMesh(core_axis_name="c", subcore_axis_name="s",
                                num_cores=1, num_subcores=1),
    compiler_params=pltpu.CompilerParams(needs_layout_passes=False),
    scratch_types=[
        pltpu.VMEM((_B,), jnp.int32),
        pltpu.VMEM((_B,), jnp.int32),
        pltpu.VMEM((_B,), jnp.float32),
        pltpu.VMEM((_HPAD,), jnp.float32),
        pltpu.VMEM((_L,), jnp.float32),
        pltpu.SemaphoreType.DMA,
    ],
)(_tec_body)


def kernel(input, target, distr):
    del distr
    out = _negloss_sc(input.reshape(-1), target)
    return out[0]


# speedup vs baseline: 1.0022x; 1.0022x over previous
import functools

import jax
import jax.numpy as jnp
from jax import lax
from jax.experimental import pallas as pl
from jax.experimental.pallas import tpu as pltpu
from jax.experimental.pallas import tpu_sc as plsc

_B = 128
_V = 1000
_L = 16
_NCH = _B // _L
_HPAD = ((_V + _L - 1) // _L) * _L


def _tec_body(inp_hbm, tgt_hbm, out_hbm, tgt_v, idx_v, picked_v, hist_v,
              out_v, sem):
    pltpu.sync_copy(tgt_hbm, tgt_v)

    zeros = jnp.zeros((_L,), jnp.float32)
    ones = jnp.ones((_L,), jnp.float32)
    iota = lax.iota(jnp.int32, _L)
    for k in range(_NCH):
        t = tgt_v[pl.ds(k * _L, _L)]
        plsc.store_scatter(hist_v, [t], zeros)
        idx_v[pl.ds(k * _L, _L)] = t + (iota + k * _L) * _V

    cp = pltpu.async_copy(inp_hbm.at[idx_v], picked_v, sem)
    for k in range(_NCH):
        t = tgt_v[pl.ds(k * _L, _L)]
        plsc.addupdate_scatter(hist_v, [t], ones)
    cp.wait()

    num = jnp.zeros((_L,), jnp.float32)
    den = jnp.zeros((_L,), jnp.float32)
    for k in range(_NCH):
        t = tgt_v[pl.ds(k * _L, _L)]
        wt = plsc.load_gather(hist_v, [t])
        num = num + wt * picked_v[pl.ds(k * _L, _L)]
        den = den + wt

    n_v = jnp.broadcast_to(jnp.sum(num), (_L,))
    d_v = jnp.broadcast_to(jnp.sum(den), (_L,))
    out_v[...] = -(n_v / d_v)
    pltpu.sync_copy(out_v, out_hbm)


_negloss_sc = functools.partial(
    pl.kernel,
    out_type=jax.ShapeDtypeStruct((_L,), jnp.float32),
    mesh=plsc.VectorSubcoreMesh(core_axis_name="c", subcore_axis_name="s",
                                num_cores=1, num_subcores=1),
    compiler_params=pltpu.CompilerParams(
        needs_layout_passes=False,
        disable_bounds_checks=True,
        disable_semaphore_checks=True,
    ),
    scratch_types=[
        pltpu.VMEM((_B,), jnp.int32),
        pltpu.VMEM((_B,), jnp.int32),
        pltpu.VMEM((_B,), jnp.float32),
        pltpu.VMEM((_HPAD,), jnp.float32),
        pltpu.VMEM((_L,), jnp.float32),
        pltpu.SemaphoreType.DMA,
    ],
)(_tec_body)


def kernel(input, target, distr):
    del distr
    out = _negloss_sc(input.reshape(-1), target)
    return out[0]
